# Initial kernel scaffold; baseline (speedup 1.0000x reference)
import jax
import jax.numpy as jnp
from jax.experimental import pallas as pl


def kernel(x, params):
    raise NotImplementedError("write your pallas kernel here")


# speedup vs baseline: 3.2308x; 3.2308x over previous
import jax
import jax.numpy as jnp
from jax.experimental import pallas as pl
from jax.experimental.pallas import tpu as pltpu

NE = 8
CAP = 32.0
NCLS = 10
PEN = 2.0 * (1.0 / NE)

TRUNK_CONV_BF16 = True
TRUNK_DOT_BF16 = True

G1, O1, R1, B1, N1, T1 = 64, 4, 228 * 64, 128, 224 * 64, 7168
G2, O2, R2, B2, N2, T2 = 32, 2, 116 * 32, 64, 112 * 32, 3584
G3, O3, R3, B3, N3, T3 = 16, 1, 60 * 16, 32, 56 * 16, 896

F32 = jnp.float32
BF16 = jnp.bfloat16


def _dot(a, b, bf):
    if bf:
        return jnp.dot(a.astype(BF16), b.astype(BF16), preferred_element_type=F32)
    return jnp.dot(a, b, preferred_element_type=F32)


def _mk_edges(ref, C, G, first, last):
    R = ref.shape[0]
    L = R - 16
    gi = jax.lax.rem(jax.lax.broadcasted_iota(jnp.int32, (L, 1), 0) + 8, G)
    srv = ref[pl.ds(7, L), :][:, 3 * C:4 * C] * (gi != first).astype(F32)
    slv = ref[pl.ds(9, L), :][:, 0:C] * (gi != last).astype(F32)
    ref[pl.ds(8, L), 4 * C:8 * C] = jnp.concatenate(
        [srv, slv, jnp.zeros((L, 2 * C), F32)], axis=1)


def _conv_p4(src, dst, G, base, nout, tile, C, Co, w, gs, gb, bf):
    wb = w.astype(BF16) if bf else w

    def body(t, _):
        b0 = pl.multiple_of(base + t * tile, 8)
        parts = [src[pl.ds(b0 + dy * G, tile), :] for dy in (-1, 0, 1)]
        a = jnp.concatenate(parts, axis=1)
        if bf:
            a = a.astype(BF16)
        acc = jnp.dot(a, wb, preferred_element_type=F32)
        dst[pl.ds(b0, tile), 0:4 * Co] = jnp.maximum(acc * gs + gb, 0.0)
        return 0

    jax.lax.fori_loop(0, nout // tile, body, 0)
    dst[pl.ds(0, base)] = jnp.zeros((base, dst.shape[1]), F32)
    tail = dst.shape[0] - base - nout
    dst[pl.ds(base + nout, tail)] = jnp.zeros((tail, dst.shape[1]), F32)


def _pool_p4(src, sbase, dst, dbase, Hy, G, C):
    PY = 28
    Go = G // 2

    def body(q, _):
        py0 = q * PY
        i0 = pl.multiple_of(sbase + 2 * py0 * G, 8)
        v = src[pl.ds(i0, 2 * PY * G), 0:4 * C]
        xm = jnp.concatenate(
            [jnp.maximum(v[:, 0:C], v[:, C:2 * C]),
             jnp.maximum(v[:, 2 * C:3 * C], v[:, 3 * C:4 * C])], axis=1)
        v4 = xm.reshape(PY, 2, G, 2 * C)
        ym = jnp.maximum(v4[:, 0], v4[:, 1])
        v5 = ym.reshape(PY, Go, 2, 2 * C)
        out = jnp.concatenate([v5[:, :, 0, :], v5[:, :, 1, :]], axis=2)
        o0 = pl.multiple_of(dbase + py0 * Go, 8)
        dst[pl.ds(o0, PY * Go), 0:4 * C] = out.reshape(PY * Go, 4 * C)
        return 0

    jax.lax.fori_loop(0, (Hy // 2) // PY, body, 0)


def _trunk_body(x_ref, w1, gs1, gb1, w2, gs2, gb2, wfc, bfc, wg1, bg1, wg2, bg2,
                out_ref, y1, y2, pt):
    cbf, dbf = TRUNK_CONV_BF16, TRUNK_DOT_BF16
    x = x_ref.at[0]
    _conv_p4(x, y1, G1, B1, N1, T1, 4, 32, w1[0], gs1[0], gb1[0], cbf)
    _mk_edges(y1, 32, G1, O1, O1 + 55)
    _conv_p4(y1, y2, G1, B1, N1, T1, 32, 32, w2[0], gs2[0], gb2[0], cbf)
    _pool_p4(y2, B1, pt, 0, 224, G1, 32)
    hs = []
    for iy in range(2):
        v = pt[pl.ds(iy * 1792, 1792)].reshape(56, 32, 128)
        tsum = jnp.sum(v, axis=0)
        for ix in range(2):
            q = jnp.sum(tsum[O2 + ix * 14:O2 + (ix + 1) * 14], axis=0,
                        keepdims=True)
            hs.append(q[:, 0:32] + q[:, 32:64] + q[:, 64:96] + q[:, 96:128])
    h = jnp.concatenate(hs, axis=1) * (1.0 / 3136.0)
    r = jnp.maximum(_dot(h, wfc[0], dbf) + bfc[0], 0.0)
    h1 = jnp.maximum(_dot(r, wg1[0], dbf) + bg1[0], 0.0)
    sc = _dot(h1, wg2[0], dbf) + bg2[0]
    out_ref[0] = sc - PEN


def _route_body(b_ref, rw_ref):
    lanes = jax.lax.broadcasted_iota(jnp.int32, (1, NE), 1)

    def body(s, loads):
        row = b_ref[pl.ds(s, 1), :]
        m0 = jnp.max(row)
        e0 = jnp.min(jnp.where(row == m0, lanes, NE))
        row1 = jnp.where(lanes == e0, -jnp.inf, row)
        m1 = jnp.max(row1)
        e1 = jnp.min(jnp.where(row1 == m1, lanes, NE))
        l0 = jnp.sum(jnp.where(lanes == e0, loads, 0.0))
        l1 = jnp.sum(jnp.where(lanes == e1, loads, 0.0))
        fb = jnp.where(l1 < l0, e1, e0)
        e = jnp.where(l0 < CAP, e0, jnp.where(l1 < CAP, e1, fb))
        onehot = (lanes == e).astype(F32)
        rw_ref[pl.ds(s, 1), :] = onehot
        return loads + onehot

    jax.lax.fori_loop(0, 64, body, jnp.zeros((1, NE), F32))


def _expert_body(eidx, x_ref, w1, gs1, gb1, w2, gs2, gb2, w3, gs3, gb3,
                 wfc, bfc, wk1, bk1, wk2, bk2, out_ref,
                 y1, x2, y2, x3, y3):
    x = x_ref.at[0]
    _conv_p4(x, y1, G1, B1, N1, T1, 4, 32, w1[0], gs1[0], gb1[0], True)
    _mk_edges(y1, 32, G1, O1, O1 + 55)
    x2[pl.ds(0, B2)] = jnp.zeros((B2, 256), F32)
    x2[pl.ds(B2 + N2, R2 - B2 - N2)] = jnp.zeros((R2 - B2 - N2, 256), F32)
    _pool_p4(y1, B1, x2, B2, 224, G1, 32)
    _mk_edges(x2, 32, G2, O2, O2 + 27)
    _conv_p4(x2, y2, G2, B2, N2, T2, 32, 64, w2[0], gs2[0], gb2[0], True)
    x3[pl.ds(0, B3)] = jnp.zeros((B3, 512), F32)
    x3[pl.ds(B3 + N3, R3 - B3 - N3)] = jnp.zeros((R3 - B3 - N3, 512), F32)
    _pool_p4(y2, B2, x3, B3, 112, G2, 64)
    _mk_edges(x3, 64, G3, O3, O3 + 13)
    _conv_p4(x3, y3, G3, B3, N3, T3, 64, 128, w3[0], gs3[0], gb3[0], True)
    v = y3[pl.ds(B3, N3)].reshape(56, 16, 512)
    q = jnp.sum(v, axis=0)
    q = jnp.sum(q[O3:O3 + 14], axis=0, keepdims=True)
    h = (q[:, 0:128] + q[:, 128:256] + q[:, 256:384] + q[:, 384:512]) \
        * (1.0 / 3136.0)
    h = jnp.maximum(_dot(h, wfc[0], True) + bfc[0], 0.0)
    h = jnp.maximum(_dot(h, wk1[0], True) + bk1[0], 0.0)
    out_ref[0] = _dot(h, wk2[0], True) + bk2[0]


def _conv_w4(w, cin_pad=None):
    if cin_pad is not None:
        w = jnp.pad(w, ((0, 0), (0, cin_pad - w.shape[1]), (0, 0), (0, 0)))
    wt = jnp.transpose(w, (2, 3, 1, 0))
    cin, cout = wt.shape[2], wt.shape[3]
    rels = [0, 1, 2, 3, -1, 4, None, None]
    wm = jnp.zeros((3, 8, cin, 4, cout), F32)
    for pidx, rel in enumerate(rels):
        if rel is None:
            continue
        for j in range(4):
            dx = rel - j + 1
            if 0 <= dx <= 2:
                wm = wm.at[:, pidx, :, j, :].set(wt[:, dx])
    return wm.reshape(24 * cin, 4 * cout)


def kernel(x, params):
    p = params
    B = x.shape[0]
    bninv = 1.0 / jnp.sqrt(jnp.float32(1.0 + 1e-5))

    xg = jnp.pad(x.transpose(0, 2, 3, 1), ((0, 0), (2, 2), (0, 0), (0, 1)))
    xg = xg.reshape(B, 228, 56, 16)
    xg = jnp.pad(xg, ((0, 0), (0, 0), (O1, G1 - 56 - O1), (0, 0)))
    srl = jnp.pad(xg[:, :, :-1, 12:16], ((0, 0), (0, 0), (1, 0), (0, 0)))
    sll = jnp.pad(xg[:, :, 1:, 0:4], ((0, 0), (0, 0), (0, 1), (0, 0)))
    x0 = jnp.concatenate(
        [xg, srl, sll, jnp.zeros(xg.shape[:3] + (8,), F32)], axis=-1)
    x0 = x0.reshape(B, R1, 32)

    def bn4(g):
        return jnp.tile(g * bninv, 4).reshape(1, -1)

    def b4(b):
        return jnp.tile(b, 4).reshape(1, -1)

    tw = [_conv_w4(p['tc1_w'], 4), bn4(p['tc1_g']), b4(p['tc1_b']),
          _conv_w4(p['tc2_w']), bn4(p['tc2_g']), b4(p['tc2_b']),
          jnp.transpose(p['t_fc_w'].reshape(64, 32, 2, 2), (2, 3, 1, 0)).reshape(128, 64),
          p['t_fc_b'].reshape(1, -1),
          p['g1_w'].T, p['g1_b'].reshape(1, -1),
          p['g2_w'].T, p['g2_b'].reshape(1, -1)]
    tspecs = [pl.BlockSpec((1, R1, 32), lambda s: (s, 0, 0))]
    tw_in = []
    for a in tw:
        a3 = a[None]
        tw_in.append(a3)
        tspecs.append(pl.BlockSpec(a3.shape, lambda s: (0, 0, 0)))

    balanced3 = pl.pallas_call(
        _trunk_body,
        grid=(B,),
        in_specs=tspecs,
        out_specs=pl.BlockSpec((1, 1, NE), lambda s: (s, 0, 0)),
        out_shape=jax.ShapeDtypeStruct((B, 1, NE), F32),
        scratch_shapes=[
            pltpu.VMEM((R1, 256), F32),
            pltpu.VMEM((R1, 128), F32),
            pltpu.VMEM((N2, 128), F32),
        ],
    )(x0, *tw_in)
    balanced = balanced3.reshape(B, NE)

    rw = pl.pallas_call(
        _route_body,
        out_shape=jax.ShapeDtypeStruct((B, NE), F32),
    )(balanced)
    e_idx = jnp.argmax(rw, axis=1).astype(jnp.int32)

    ew = [
        jax.vmap(lambda w: _conv_w4(w, 4))(p['e_c1_w']),
        jax.vmap(bn4)(p['e_c1_g']), jax.vmap(b4)(p['e_c1_b']),
        jax.vmap(_conv_w4)(p['e_c2_w']),
        jax.vmap(bn4)(p['e_c2_g']), jax.vmap(b4)(p['e_c2_b']),
        jax.vmap(_conv_w4)(p['e_c3_w']),
        jax.vmap(bn4)(p['e_c3_g']), jax.vmap(b4)(p['e_c3_b']),
        jnp.transpose(p['e_fc_w'], (0, 2, 1)), p['e_fc_b'][:, None, :],
        jnp.transpose(p['e_cls1_w'], (0, 2, 1)), p['e_cls1_b'][:, None, :],
        jnp.pad(jnp.transpose(p['e_cls2_w'], (0, 2, 1)), ((0, 0), (0, 0), (0, 6))),
        jnp.pad(p['e_cls2_b'][:, None, :], ((0, 0), (0, 0), (0, 6))),
    ]
    especs = [pl.BlockSpec((1, R1, 32), lambda s, ei: (s, 0, 0))]
    for a in ew:
        especs.append(pl.BlockSpec((1,) + a.shape[1:],
                                   lambda s, ei: (ei[s], 0, 0)))

    logits3 = pl.pallas_call(
        _expert_body,
        grid_spec=pltpu.PrefetchScalarGridSpec(
            num_scalar_prefetch=1,
            grid=(B,),
            in_specs=especs,
            out_specs=pl.BlockSpec((1, 1, 16), lambda s, ei: (s, 0, 0)),
            scratch_shapes=[
                pltpu.VMEM((R1, 256), F32),
                pltpu.VMEM((R2, 256), F32),
                pltpu.VMEM((R2, 256), F32),
                pltpu.VMEM((R3, 512), F32),
                pltpu.VMEM((R3, 512), F32),
            ],
        ),
        out_shape=jax.ShapeDtypeStruct((B, 1, 16), F32),
    )(e_idx, x0, *ew)
    out = logits3[:, 0, :NCLS]
    return out, rw, balanced
